# baseline (speedup 1.0000x reference)
import jax
import jax.numpy as jnp
from jax.experimental import pallas as pl
from jax.experimental.pallas import tpu as pltpu

N, D, H, E = 8192, 1024, 512, 64
BN = 2048
R = 512
NSUB = BN // R
NB = N // BN


def _copy(x_hbm, xbuf, sem, chunk, slot):
    return pltpu.make_async_copy(
        x_hbm.at[pl.ds(chunk * R, R), :], xbuf.at[slot], sem.at[slot])


def _fused_kernel(x_hbm, w0_ref, b0_ref, w1_ref, b1_ref, w2_ref, b2_ref,
                  w3_ref, b3_ref, soft_ref, hard_ref, xbuf, sem):
    i = pl.program_id(0)
    w0, w1, w2, w3 = w0_ref[...], w1_ref[...], w2_ref[...], w3_ref[...]
    b0, b1, b2, b3 = b0_ref[...], b1_ref[...], b2_ref[...], b3_ref[...]

    @pl.when(i == 0)
    def _bootstrap():
        _copy(x_hbm, xbuf, sem, 0, 0).start()

    for s in range(NSUB):
        g = i * NSUB + s
        if s < NSUB - 1:
            _copy(x_hbm, xbuf, sem, g + 1, (s + 1) % 2).start()
        else:
            @pl.when(i < NB - 1)
            def _prefetch_next_block():
                _copy(x_hbm, xbuf, sem, g + 1, 0).start()
        _copy(x_hbm, xbuf, sem, g, s % 2).wait()

        rows = pl.ds(s * R, R)
        x = xbuf[s % 2]
        h = jnp.maximum(jnp.dot(x, w0, preferred_element_type=jnp.float32) + b0, 0.0)
        h = jnp.maximum(jnp.dot(h, w1, preferred_element_type=jnp.float32) + b1, 0.0)
        h = jnp.maximum(jnp.dot(h, w2, preferred_element_type=jnp.float32) + b2, 0.0)
        logits = jnp.dot(h, w3, preferred_element_type=jnp.float32) + b3

        m = jnp.max(logits, axis=-1, keepdims=True)
        e = jnp.exp(logits - m)
        ssum = jnp.sum(e, axis=-1, keepdims=True)
        soft = e / ssum
        soft_ref[rows, :] = soft

        cols = jax.lax.broadcasted_iota(jnp.int32, soft.shape, 1)
        sm = 1.0 / ssum
        idx = jnp.min(jnp.where(soft == sm, cols, E), axis=-1, keepdims=True)
        hard_ref[rows, :] = (cols == idx).astype(jnp.float32)


def kernel(x_z, W0, b0, W1, b1, W2, b2, W3, b3):
    grid = (NB,)
    full = lambda a: pl.BlockSpec(a.shape, lambda i: (0,) * a.ndim)
    b0r, b1r, b2r, b3r = (b.reshape(1, -1) for b in (b0, b1, b2, b3))
    out_spec = pl.BlockSpec((BN, E), lambda i: (i, 0))
    soft, hard = pl.pallas_call(
        _fused_kernel,
        grid=grid,
        in_specs=[pl.BlockSpec(memory_space=pl.ANY),
                  full(W0), full(b0r), full(W1), full(b1r),
                  full(W2), full(b2r), full(W3), full(b3r)],
        out_specs=[out_spec, out_spec],
        out_shape=[jax.ShapeDtypeStruct((N, E), jnp.float32)] * 2,
        scratch_shapes=[pltpu.VMEM((2, R, D), jnp.float32),
                        pltpu.SemaphoreType.DMA((2,))],
        compiler_params=pltpu.CompilerParams(
            dimension_semantics=("arbitrary",),
        ),
    )(x_z, W0, b0r, W1, b1r, W2, b2r, W3, b3r)
    return (soft[..., None], hard[..., None])


# speedup vs baseline: 1.1896x; 1.1896x over previous
import jax
import jax.numpy as jnp
from jax.experimental import pallas as pl
from jax.experimental.pallas import tpu as pltpu

N, D, H, E = 8192, 1024, 512, 64
BN = 2048
SUBS = (512, 512, 512, 512)


def _fused_kernel(x_ref, w0_ref, b0_ref, w1_ref, b1_ref, w2_ref, b2_ref,
                  w3_ref, b3_ref, soft_ref, hard_ref):
    w0, w1, w2, w3 = w0_ref[...], w1_ref[...], w2_ref[...], w3_ref[...]
    b0, b1, b2, b3 = b0_ref[...], b1_ref[...], b2_ref[...], b3_ref[...]
    base = 0
    for r in SUBS:
        rows = pl.ds(base, r)
        base += r
        x = x_ref[rows, :]
        h = jnp.maximum(jnp.dot(x, w0, preferred_element_type=jnp.float32) + b0, 0.0)
        h = jnp.maximum(jnp.dot(h, w1, preferred_element_type=jnp.float32) + b1, 0.0)
        h = jnp.maximum(jnp.dot(h, w2, preferred_element_type=jnp.float32) + b2, 0.0)
        logits = jnp.dot(h, w3, preferred_element_type=jnp.float32) + b3

        m = jnp.max(logits, axis=-1, keepdims=True)
        e = jnp.exp(logits - m)
        ssum = jnp.sum(e, axis=-1, keepdims=True)
        soft = e / ssum
        soft_ref[rows, :] = soft

        cols = jax.lax.broadcasted_iota(jnp.int32, soft.shape, 1)
        sm = 1.0 / ssum
        idx = jnp.min(jnp.where(soft == sm, cols, E), axis=-1, keepdims=True)
        hard_ref[rows, :] = (cols == idx).astype(jnp.float32)


def kernel(x_z, W0, b0, W1, b1, W2, b2, W3, b3):
    grid = (N // BN,)
    row_spec = pl.BlockSpec((BN, D), lambda i: (i, 0))
    full = lambda a: pl.BlockSpec(a.shape, lambda i: (0,) * a.ndim)
    b0r, b1r, b2r, b3r = (b.reshape(1, -1) for b in (b0, b1, b2, b3))
    out_spec = pl.BlockSpec((BN, E), lambda i: (i, 0))
    soft, hard = pl.pallas_call(
        _fused_kernel,
        grid=grid,
        in_specs=[row_spec, full(W0), full(b0r), full(W1), full(b1r),
                  full(W2), full(b2r), full(W3), full(b3r)],
        out_specs=[out_spec, out_spec],
        out_shape=[jax.ShapeDtypeStruct((N, E), jnp.float32)] * 2,
        compiler_params=pltpu.CompilerParams(
            dimension_semantics=("arbitrary",),
        ),
    )(x_z, W0, b0r, W1, b1r, W2, b2r, W3, b3r)
    return (soft[..., None], hard[..., None])
